# baseline (speedup 1.0000x reference)
import functools
import math

import jax
import jax.numpy as jnp
from jax.experimental import pallas as pl

_CONV_WINDOW = 5
_EPS = 1e-12

_ACOS_COEFFS = tuple(
    c / math.pi for c in (1.5707288, -0.2121144, 0.0742610, -0.0187293)
)


def _collision_prob(x):
    a = jnp.abs(x)
    poly = _ACOS_COEFFS[-1]
    for c in reversed(_ACOS_COEFFS[:-1]):
        poly = poly * a + c
    y = 1.0 - a
    v = y * jax.lax.rsqrt(y) * poly
    return jnp.where(x >= 0, 1.0 - v, v)


def _l2n(x):
    ss = jnp.sum(x * x, axis=-1, keepdims=True)
    return x * jax.lax.rsqrt(jnp.maximum(ss, _EPS * _EPS))


def _yoso_head(q_ref, k_ref, vp_ref, m_ref, w_ref, o_ref, *, ts, seq, pad):
    qn = _l2n(q_ref[0])
    kn = _l2n(k_ref[0])
    vm = vp_ref[0, pad:pad + seq, :]
    w = w_ref[0, 0]

    qb = qn.astype(jnp.bfloat16)
    kb = kn.astype(jnp.bfloat16)
    vb = vm.astype(jnp.bfloat16)
    for i in range(seq // ts):
        r0 = i * ts
        qt = qb[r0:r0 + ts, :]
        qk = jax.lax.dot_general(qt, kb, (((1,), (1,)), ((), ())),
                                 preferred_element_type=jnp.float32)
        qk = jnp.clip(qk, -1.0 + 1e-6, 1.0 - 1e-6)
        p = _collision_prob(qk)
        p2 = p * p
        p4 = p2 * p2
        p8 = p4 * p4
        x = jax.lax.dot_general(p8.astype(jnp.bfloat16), vb,
                                (((1,), (0,)), ((), ())),
                                preferred_element_type=jnp.float32)
        x = x * m_ref[0, r0:r0 + ts][:, None]
        x = _l2n(x)
        conv = x
        for j in range(_CONV_WINDOW):
            lo = r0 + j + pad - _CONV_WINDOW // 2
            conv = conv + vp_ref[0, lo:lo + ts, :] * w[j]
        o_ref[0, r0:r0 + ts, :] = conv


def kernel(Q, K, V, mask, W_conv):
    B, H, S, D = Q.shape
    BH = B * H
    TS = 256
    PAD = 4

    Qf = Q.reshape(BH, S, D)
    Kf = K.reshape(BH, S, D)
    Vm = (V * mask[:, None, :, None]).reshape(BH, S, D)
    Vp = jnp.pad(Vm, ((0, 0), (PAD, PAD), (0, 0)))
    Wc = W_conv.reshape(H, 1, _CONV_WINDOW)

    out = pl.pallas_call(
        functools.partial(_yoso_head, ts=TS, seq=S, pad=PAD),
        grid=(BH,),
        in_specs=[
            pl.BlockSpec((1, S, D), lambda bh: (bh, 0, 0)),
            pl.BlockSpec((1, S, D), lambda bh: (bh, 0, 0)),
            pl.BlockSpec((1, S + 2 * PAD, D), lambda bh: (bh, 0, 0)),
            pl.BlockSpec((1, S), lambda bh: (bh // H, 0)),
            pl.BlockSpec((1, 1, _CONV_WINDOW), lambda bh: (bh % H, 0, 0)),
        ],
        out_specs=pl.BlockSpec((1, S, D), lambda bh: (bh, 0, 0)),
        out_shape=jax.ShapeDtypeStruct((BH, S, D), jnp.float32),
    )(Qf, Kf, Vp, mask, Wc)
    return out.reshape(B, H, S, D)


# speedup vs baseline: 1.1453x; 1.1453x over previous
import functools
import math

import jax
import jax.numpy as jnp
from jax.experimental import pallas as pl
from jax.experimental.pallas import tpu as pltpu

_CONV_WINDOW = 5
_EPS = 1e-12

_ACOS_COEFFS = tuple(
    c / math.pi for c in (1.5707288, -0.2121144, 0.0742610, -0.0187293)
)


def _collision_prob(x):
    a = jnp.minimum(jnp.abs(x), 1.0 - 1e-6)
    poly = _ACOS_COEFFS[-1]
    for c in reversed(_ACOS_COEFFS[:-1]):
        poly = poly * a + c
    y = 1.0 - a
    v = y * jax.lax.rsqrt(y) * poly
    return jnp.where(x >= 0, 1.0 - v, v)


def _l2n(x):
    ss = jnp.sum(x * x, axis=-1, keepdims=True)
    return x * jax.lax.rsqrt(jnp.maximum(ss, _EPS * _EPS))


def _yoso_head(q_ref, k_ref, vp_ref, m_ref, w_ref, o_ref, *, ts, seq, pad):
    qn = _l2n(q_ref[0])
    kn = _l2n(k_ref[0])
    vm = vp_ref[0, pad:pad + seq, :]
    w = w_ref[0, 0]

    for i in range(seq // ts):
        r0 = i * ts
        qt = qn[r0:r0 + ts, :]
        qk = jax.lax.dot_general(qt, kn, (((1,), (1,)), ((), ())),
                                 preferred_element_type=jnp.float32)
        p = _collision_prob(qk)
        p2 = p * p
        p4 = p2 * p2
        p8 = p4 * p4
        x = jax.lax.dot_general(p8, vm, (((1,), (0,)), ((), ())),
                                preferred_element_type=jnp.float32)
        x = x * m_ref[0, r0:r0 + ts][:, None]
        x = _l2n(x)
        conv = x
        for j in range(_CONV_WINDOW):
            lo = r0 + j + pad - _CONV_WINDOW // 2
            conv = conv + vp_ref[0, lo:lo + ts, :] * w[j]
        o_ref[0, r0:r0 + ts, :] = conv


def kernel(Q, K, V, mask, W_conv):
    B, H, S, D = Q.shape
    BH = B * H
    TS = 256
    PAD = 4

    Qf = Q.reshape(BH, S, D)
    Kf = K.reshape(BH, S, D)
    Vm = (V * mask[:, None, :, None]).reshape(BH, S, D)
    Vp = jnp.pad(Vm, ((0, 0), (PAD, PAD), (0, 0)))
    Wc = W_conv.reshape(H, 1, _CONV_WINDOW)

    out = pl.pallas_call(
        functools.partial(_yoso_head, ts=TS, seq=S, pad=PAD),
        grid=(BH,),
        in_specs=[
            pl.BlockSpec((1, S, D), lambda bh: (bh, 0, 0)),
            pl.BlockSpec((1, S, D), lambda bh: (bh, 0, 0)),
            pl.BlockSpec((1, S + 2 * PAD, D), lambda bh: (bh, 0, 0)),
            pl.BlockSpec((1, S), lambda bh: (bh // H, 0)),
            pl.BlockSpec((1, 1, _CONV_WINDOW), lambda bh: (bh % H, 0, 0)),
        ],
        out_specs=pl.BlockSpec((1, S, D), lambda bh: (bh, 0, 0)),
        out_shape=jax.ShapeDtypeStruct((BH, S, D), jnp.float32),
        compiler_params=pltpu.CompilerParams(
            dimension_semantics=("parallel",)),
    )(Qf, Kf, Vp, mask, Wc)
    return out.reshape(B, H, S, D)
